# baseline (speedup 1.0000x reference)
import functools

import jax
import jax.numpy as jnp
from jax import lax
from jax.experimental import pallas as pl
from jax.experimental.pallas import tpu as pltpu
from jax.experimental.pallas import tpu_sc as plsc

_TOPK = 25
_SKIP = 3
_NSEL = _TOPK - _SKIP
_B = 64
_C = 1000
_HW = 28 * 28
_CPAD = 1024
_NVREG = _CPAD // 16
_NC = 2
_NS = 16
_BPW = _B // (_NC * _NS)
_NEG = float("-inf")
_BIG = 1 << 30


def _sc_body(p_hbm, mo_hbm, feat_hbm, out_hbm,
             mo_v, idx_v, rows_v, p_v, tot_v, big_v, shared, sem):
    cid = lax.axis_index("c")
    sid = lax.axis_index("s")
    wid = cid * _NS + sid
    b0 = wid * _BPW
    lanes = lax.iota(jnp.int32, 16)
    lane0 = lanes == 0

    for bi in range(_BPW):
        b = b0 + bi
        pltpu.sync_copy(mo_hbm.at[pl.ds(b * _C, _C)], mo_v.at[pl.ds(0, _C)])
        tail = mo_v[pl.ds(_CPAD - 32, 16)]
        mo_v[pl.ds(_CPAD - 32, 16)] = jnp.where(lanes < 8, tail, _NEG)
        mo_v[pl.ds(_CPAD - 16, 16)] = jnp.full((16,), _NEG, jnp.float32)

        def init_step(k, carry):
            m, mi = carry
            v = mo_v[pl.ds(k * 16, 16)]
            upd = v > m
            return jnp.where(upd, v, m), jnp.where(upd, k, mi)

        m0 = jnp.full((16,), _NEG, jnp.float32)
        mi0 = jnp.zeros((16,), jnp.int32)
        M, MI = lax.fori_loop(0, _NVREG, init_step, (m0, mi0))

        def topk_step(t, carry):
            m, mi = carry
            mx = jnp.max(m)
            cand = jnp.where(m == mx, mi * 16 + lanes, _BIG)
            row_idx = jnp.min(cand)
            pos = bi * _NSEL + jnp.maximum(t - _SKIP, 0)
            gid = b * _C + row_idx
            plsc.store_scatter(idx_v, [jnp.full((16,), pos)],
                               jnp.full((16,), gid),
                               mask=lane0 & (t >= _SKIP))
            plsc.store_scatter(mo_v, [jnp.full((16,), row_idx)],
                               jnp.full((16,), _NEG, jnp.float32), mask=lane0)
            col = lax.rem(row_idx, 16)
            vm = jnp.full((16,), _NEG, jnp.float32)
            km = jnp.zeros((16,), jnp.int32)
            for j in range(_NVREG // 16):
                g = plsc.load_gather(mo_v, [col + 256 * j + 16 * lanes])
                upd = g > vm
                vm = jnp.where(upd, g, vm)
                km = jnp.where(upd, j, km)
            cmx = jnp.max(vm)
            ck = jnp.where(vm == cmx, km * 16 + lanes, _BIG)
            kmin = jnp.min(ck)
            hit = lanes == col
            return jnp.where(hit, cmx, m), jnp.where(hit, kmin, mi)

        lax.fori_loop(0, _TOPK, topk_step, (M, MI))

    pltpu.async_copy(feat_hbm.at[idx_v], rows_v, sem).wait()
    pltpu.sync_copy(p_hbm.at[pl.ds(b0, _BPW)], p_v)

    def row_sum(r, acc):
        for s in range(_HW // 16):
            acc = acc + rows_v[r, pl.ds(s * 16, 16)]
        return acc

    tot = lax.fori_loop(0, _BPW * _NSEL, row_sum,
                        jnp.zeros((16,), jnp.float32))
    for bi in range(_BPW):
        for s in range(_HW // 16):
            tot = tot + p_v[bi, pl.ds(s * 16, 16)]
    tot_v[...] = tot * jnp.float32(1.0 / (_B * _HW))

    pltpu.sync_copy(tot_v, shared.at[sid])
    plsc.subcore_barrier()

    @pl.when(sid == 0)
    def _():
        pltpu.sync_copy(shared, big_v)
        acc = big_v[0, pl.ds(0, 16)]
        for r in range(1, _NS):
            acc = acc + big_v[r, pl.ds(0, 16)]
        tot_v[...] = acc
        pltpu.sync_copy(tot_v, out_hbm.at[cid])


@functools.partial(
    pl.kernel,
    out_type=jax.ShapeDtypeStruct((_NC, 16), jnp.float32),
    mesh=plsc.VectorSubcoreMesh(core_axis_name="c", subcore_axis_name="s"),
    compiler_params=pltpu.CompilerParams(needs_layout_passes=False,
                                         use_tc_tiling_on_sc=False),
    scratch_types=[
        pltpu.VMEM((_CPAD,), jnp.float32),
        pltpu.VMEM((_BPW * _NSEL,), jnp.int32),
        pltpu.VMEM((_BPW * _NSEL, _HW), jnp.float32),
        pltpu.VMEM((_BPW, _HW), jnp.float32),
        pltpu.VMEM((16,), jnp.float32),
        pltpu.VMEM((_NS, 16), jnp.float32),
        pltpu.VMEM_SHARED((_NS, 16), jnp.float32),
        pltpu.SemaphoreType.DMA,
    ],
)
def _area_loss_sc(p_hbm, mo_hbm, feat_hbm, out_hbm, *scratch):
    _sc_body(p_hbm, mo_hbm, feat_hbm, out_hbm, *scratch)


def kernel(p, main_out, features):
    p2 = p.reshape(_B, p.shape[2] * p.shape[3])
    mo = main_out.reshape(-1)
    feat2 = features.reshape(_B * _C, _HW)
    partials = _area_loss_sc(p2, mo, feat2)
    return jnp.sum(partials)


# speedup vs baseline: 4.8347x; 4.8347x over previous
import functools

import jax
import jax.numpy as jnp
from jax import lax
from jax.experimental import pallas as pl
from jax.experimental.pallas import tpu as pltpu
from jax.experimental.pallas import tpu_sc as plsc

_TOPK = 25
_SKIP = 3
_NSEL = _TOPK - _SKIP
_B = 64
_C = 1000
_HW = 28 * 28
_NC = 2
_NS = 16
_HB = _B // _NC
_CHUNKS = _HW // _NS
_NEG = float("-inf")
_BIG = 1 << 30
_STG = 128


def _topk_row(mo8_v, stage_v, r, bi, lanes, lane0):
    def init_step(k, carry):
        m, mi = carry
        e = k * 16 + lanes
        inb = e < _C
        g = plsc.load_gather(mo8_v, [jnp.full((16,), r), e], mask=inb)
        v = jnp.where(inb, g, _NEG)
        upd = v > m
        return jnp.where(upd, v, m), jnp.where(upd, k, mi)

    m0 = jnp.full((16,), _NEG, jnp.float32)
    mi0 = jnp.zeros((16,), jnp.int32)
    M, MI = lax.fori_loop(0, 64, init_step, (m0, mi0))

    def topk_step(t, carry):
        m, mi = carry
        mx = jnp.max(m)
        cand = jnp.where(m == mx, mi * 16 + lanes, _BIG)
        c_sel = jnp.minimum(jnp.min(cand), _C - 1)
        pos = 64 + bi * _NSEL + jnp.maximum(t - _SKIP, 0)
        plsc.store_scatter(stage_v, [jnp.full((16,), pos)],
                           jnp.full((16,), c_sel),
                           mask=lane0 & (t >= _SKIP))
        plsc.store_scatter(mo8_v, [jnp.full((16,), r), jnp.full((16,), c_sel)],
                           jnp.full((16,), _NEG, jnp.float32), mask=lane0)
        col = lax.rem(c_sel, 16)
        vm = jnp.full((16,), _NEG, jnp.float32)
        km = jnp.zeros((16,), jnp.int32)
        for j in range(4):
            e = col + 256 * j + 16 * lanes
            inb = e < _C
            g = plsc.load_gather(mo8_v, [jnp.full((16,), r), e], mask=inb)
            v = jnp.where(inb, g, _NEG)
            upd = v > vm
            vm = jnp.where(upd, v, vm)
            km = jnp.where(upd, j, km)
        cmx = jnp.max(vm)
        ck = jnp.where(vm == cmx, km * 16 + lanes, _BIG)
        kmin = jnp.min(ck)
        hit = lanes == col
        return jnp.where(hit, cmx, m), jnp.where(hit, kmin, mi)

    lax.fori_loop(0, _TOPK, topk_step, (M, MI))


def _sc_body(p_hbm, mo_hbm, tf_hbm, out_hbm,
             mo8_v, stage_v, allidx_v, buf0, p8_v, tot_v,
             sh_idx, sem0, sem1):
    cid = lax.axis_index("c")
    sid = lax.axis_index("s")
    b0 = _HB * cid + 2 * sid
    grp = lax.div(b0, 8) * 8
    lanes = lax.iota(jnp.int32, 16)
    lane0 = lanes == 0

    bh0 = 2 * sid
    for k in range(_STG // 16):
        stage_v[pl.ds(16 * k, 16)] = jnp.zeros((16,), jnp.int32)
    stage_v[pl.ds(0, 16)] = jnp.full((16,), bh0)
    stage_v[pl.ds(16, 16)] = jnp.where(lanes < 6, bh0, bh0 + 1)
    stage_v[pl.ds(32, 16)] = jnp.where(lanes < 12, bh0 + 1, 0)

    pltpu.sync_copy(mo_hbm.at[pl.ds(grp, 8)], mo8_v)
    for bi in range(2):
        _topk_row(mo8_v, stage_v, b0 + bi - grp, bi, lanes, lane0)

    pltpu.sync_copy(stage_v, sh_idx.at[sid])
    plsc.subcore_barrier()
    pltpu.sync_copy(sh_idx, allidx_v)

    pltpu.sync_copy(p_hbm.at[pl.ds(grp, 8)], p8_v)
    tot = jnp.zeros((16,), jnp.float32)

    def p_step(k, acc):
        a = acc
        for bi in range(2):
            g = plsc.load_gather(
                p8_v, [jnp.full((16,), b0 + bi - grp), k * 16 + lanes])
            a = a + g
        return a

    tot = lax.fori_loop(0, _HW // 16, p_step, tot)

    half = pl.ds(_HB * cid, _HB)
    base = sid * _CHUNKS

    def consume(buf, acc):
        def row_step(rw, a):
            rs = jnp.full((16,), rw)
            for k in range(3):
                msk = (k * 16 + lanes) < 44
                idxb = plsc.load_gather(allidx_v, [rs, k * 16 + lanes])
                idxc = plsc.load_gather(allidx_v, [rs, 64 + k * 16 + lanes])
                g = plsc.load_gather(buf, [idxb, idxc], mask=msk)
                a = a + jnp.where(msk, g, 0.0)
            return a

        return lax.fori_loop(0, _NS, row_step, acc)

    def chunk_step(i, acc):
        pltpu.sync_copy(tf_hbm.at[base + i, half], buf0)
        return consume(buf0, acc)

    tot = lax.fori_loop(0, _CHUNKS, chunk_step, tot)

    tot_v[...] = tot * jnp.float32(1.0 / (_B * _HW))
    pltpu.sync_copy(tot_v, out_hbm.at[cid, sid])


@functools.partial(
    pl.kernel,
    out_type=jax.ShapeDtypeStruct((_NC, _NS, 16), jnp.float32),
    mesh=plsc.VectorSubcoreMesh(core_axis_name="c", subcore_axis_name="s"),
    compiler_params=pltpu.CompilerParams(needs_layout_passes=False),
    scratch_types=[
        pltpu.VMEM((8, _C), jnp.float32),
        pltpu.VMEM((_STG,), jnp.int32),
        pltpu.VMEM((_NS, _STG), jnp.int32),
        pltpu.VMEM((_HB, _C), jnp.float32),
        pltpu.VMEM((8, _HW), jnp.float32),
        pltpu.VMEM((16,), jnp.float32),
        pltpu.VMEM_SHARED((_NS, _STG), jnp.int32),
        pltpu.SemaphoreType.DMA,
        pltpu.SemaphoreType.DMA,
    ],
)
def _area_loss_sc(p_hbm, mo_hbm, tf_hbm, out_hbm, *scratch):
    _sc_body(p_hbm, mo_hbm, tf_hbm, out_hbm, *scratch)


def kernel(p, main_out, features):
    p2 = p.reshape(_B, _HW)
    tf = jnp.transpose(features, (2, 3, 0, 1)).reshape(_HW, _B, _C)
    partials = _area_loss_sc(p2, main_out, tf)
    return jnp.sum(partials)
